# baseline (speedup 1.0000x reference)
import functools

import jax
import jax.numpy as jnp
from jax import lax
from jax.experimental import pallas as pl
from jax.experimental.pallas import tpu as pltpu
from jax.experimental.pallas import tpu_sc as plsc

_N = 10000
_D = 128
_DA = 144
_B = 1024
_NC = 2
_NS = 16
_NW = _NC * _NS
_CH = 128
_NROWS = 10112
_SLAB = _NROWS // _NS
_G = 2 * _B


_NBUF = 2


def _agg_kernel_body(full, chunks_pw, x_hbm, src_hbm, dst_hbm, zeros_hbm,
                     gidx_hbm, out_hbm, src_i, dst_i, *bufs):
    rows = bufs[:_NBUF]
    acc = bufs[_NBUF]
    sems = bufs[_NBUF + 1:]
    c = lax.axis_index("c")
    s = lax.axis_index("s")
    wid = s * _NC + c

    z0 = pl.multiple_of(s * _SLAB, 8)
    pltpu.sync_copy(zeros_hbm.at[pl.ds(z0, _SLAB)], acc.at[pl.ds(z0, _SLAB)])
    plsc.subcore_barrier()

    groups = chunks_pw // _NBUF

    def body(g, carry):
        r0 = pl.multiple_of(wid * chunks_pw + g * _NBUF, _NBUF)
        pltpu.sync_copy(src_hbm.at[pl.ds(r0, _NBUF)], src_i)
        pltpu.sync_copy(dst_hbm.at[pl.ds(r0, _NBUF)], dst_i)
        descs = [pltpu.async_copy(x_hbm.at[src_i.at[b]], rows[b], sems[b])
                 for b in range(_NBUF)]
        for b in range(_NBUF):
            descs[b].wait()
            pltpu.sync_copy(rows[b], acc.at[dst_i.at[b]], add=True)
        return carry

    lax.fori_loop(0, groups, body, 0)
    plsc.subcore_barrier()

    if full:
        for j in range(5):
            cnt = _CH if j < 4 else _SLAB - 4 * _CH
            r0 = pl.multiple_of(s * _SLAB + j * _CH, 8)
            o0 = pl.multiple_of(c * _NROWS + s * _SLAB + j * _CH, 8)
            rb = rows[j % 2]
            pltpu.sync_copy(acc.at[pl.ds(r0, cnt)], rb.at[pl.ds(0, cnt)])
            pltpu.sync_copy(rb.at[pl.ds(0, cnt)], out_hbm.at[pl.ds(o0, cnt)])
    else:
        g0 = pl.multiple_of(s * _CH, 8)
        pltpu.sync_copy(gidx_hbm.at[pl.ds(g0, _CH)], src_i.at[0])
        pltpu.async_copy(acc.at[src_i.at[0]], rows[0], sems[0]).wait()
        o0 = pl.multiple_of((c * _NS + s) * _CH, 8)
        pltpu.sync_copy(rows[0], out_hbm.at[pl.ds(o0, _CH)])


@functools.lru_cache(maxsize=None)
def _make_agg(chunks_pw, full):
    mesh = plsc.VectorSubcoreMesh(core_axis_name="c", subcore_axis_name="s")
    out_rows = _NC * _NROWS if full else _NC * _G
    body = functools.partial(_agg_kernel_body, full, chunks_pw)
    if full:
        def wrapped(x, src, dst, zeros, out, *scratch):
            body(x, src, dst, zeros, None, out, *scratch)
    else:
        wrapped = body
    return pl.kernel(
        wrapped,
        out_type=jax.ShapeDtypeStruct((out_rows, _DA), jnp.float32),
        mesh=mesh,
        compiler_params=pltpu.CompilerParams(use_tc_tiling_on_sc=False),
        scratch_types=(
            [pltpu.VMEM((_NBUF, _CH), jnp.int32),
             pltpu.VMEM((_NBUF, _CH), jnp.int32)]
            + [pltpu.VMEM((_CH, _DA), jnp.float32)] * _NBUF
            + [pltpu.VMEM_SHARED((_NROWS, _DA), jnp.float32)]
            + [pltpu.SemaphoreType.DMA] * _NBUF
        ),
    )


_H1_BLK = 632


def _h1_body(a_ref, w_ref, o_ref):
    x = a_ref[0] + a_ref[1]
    deg = jnp.maximum(x[:, _D:_D + 1], 1.0)
    h = jnp.dot(x[:, :_D] / deg, w_ref[...], preferred_element_type=jnp.float32)
    h = jnp.maximum(h, 0.0)
    col = lax.broadcasted_iota(jnp.int32, (_H1_BLK, _DA - _D), 1)
    aug = jnp.where(col == 0, 1.0, 0.0)
    o_ref[...] = jnp.concatenate([h, aug], axis=1)


def _h1_call(a, w):
    grid = _NROWS // _H1_BLK
    return pl.pallas_call(
        _h1_body,
        grid=(grid,),
        in_specs=[
            pl.BlockSpec((_NC, _H1_BLK, _DA), lambda i: (0, i, 0)),
            pl.BlockSpec((_D, _D), lambda i: (0, 0)),
        ],
        out_specs=pl.BlockSpec((_H1_BLK, _DA), lambda i: (i, 0)),
        out_shape=jax.ShapeDtypeStruct((_NROWS, _DA), jnp.float32),
    )(a, w)


def _loss_body(p1_ref, p2_ref, w_ref, o_ref):
    def emb(p_ref):
        r = p_ref[0] + p_ref[1]
        deg = jnp.maximum(r[:, _D:_D + 1], 1.0)
        return jnp.dot(r[:, :_D] / deg, w_ref[...],
                       preferred_element_type=jnp.float32)

    e1 = emb(p1_ref)
    e2 = emb(p2_ref)
    pos1, neg1 = e1[:_B], e1[_B:]
    pos2, neg2 = e2[:_B], e2[_B:]
    pd = jnp.sum(jnp.abs(pos1 - pos2), axis=1, keepdims=True)
    na = jnp.sum(jnp.abs(pos1 - neg2), axis=1, keepdims=True)
    nb = jnp.sum(jnp.abs(neg1 - pos2), axis=1, keepdims=True)
    la = jnp.maximum(pd - na + 3.0, 0.0)
    lb = jnp.maximum(pd - nb + 3.0, 0.0)
    o_ref[0, 0] = (jnp.sum(la) + jnp.sum(lb)) / _B


def _loss_call(p1, p2, w):
    return pl.pallas_call(
        _loss_body,
        out_specs=pl.BlockSpec(memory_space=pltpu.SMEM),
        out_shape=jax.ShapeDtypeStruct((1, 1), jnp.float32),
    )(p1, p2, w)


def kernel(link, neg1, neg2, edge_index1, edge_index2, emb_table1, emb_table2,
           W1, W2):
    i32 = jnp.int32
    seed1 = link[:, 0].astype(i32)
    seed2 = link[:, 1].astype(i32)
    E = edge_index1.shape[1]
    chunks_pw = -(-E // (_NW * _CH))
    chunks_pw = -(-chunks_pw // _NBUF) * _NBUF
    e_pad = _NW * _CH * chunks_pw - E

    def prep_edges(ei):
        src = jnp.concatenate([ei[0].astype(i32), jnp.zeros((e_pad,), i32)])
        dst = jnp.concatenate([ei[1].astype(i32), jnp.full((e_pad,), _N, i32)])
        return src.reshape(-1, _CH), dst.reshape(-1, _CH)

    src1, dst1 = prep_edges(edge_index1)
    src2, dst2 = prep_edges(edge_index2)
    zeros = jnp.zeros((_NROWS, _DA), jnp.float32)

    def aug_table(t):
        return (jnp.zeros((_NROWS, _DA), jnp.float32)
                .at[:_N, :_D].set(t.astype(jnp.float32))
                .at[:_N, _D].set(1.0))

    x1 = aug_table(emb_table1)
    x2 = aug_table(emb_table2)
    gidx1 = jnp.concatenate([seed1, neg1.astype(i32)])
    gidx2 = jnp.concatenate([seed2, neg2.astype(i32)])

    agg_full = _make_agg(chunks_pw, True)
    agg_gather = _make_agg(chunks_pw, False)

    a1 = agg_full(x1, src1, dst1, zeros).reshape(_NC, _NROWS, _DA)
    a2 = agg_full(x2, src2, dst2, zeros).reshape(_NC, _NROWS, _DA)
    h1 = _h1_call(a1, W1)
    h2 = _h1_call(a2, W1)
    p1 = agg_gather(h1, src1, dst1, zeros, gidx1).reshape(_NC, _G, _DA)
    p2 = agg_gather(h2, src2, dst2, zeros, gidx2).reshape(_NC, _G, _DA)
    return _loss_call(p1, p2, W2)[0, 0]


# speedup vs baseline: 3.2425x; 3.2425x over previous
import functools

import jax
import jax.numpy as jnp
from jax import lax
from jax.experimental import pallas as pl
from jax.experimental.pallas import tpu as pltpu
from jax.experimental.pallas import tpu_sc as plsc

_N = 10000
_D = 128
_DA = 160
_B = 1024
_NC = 2
_NS = 16
_CH = 128
_NROWS = 10112
_SLAB = _NROWS // _NS
_G = 2 * _B
_NBUF = 2
_DT = jnp.bfloat16


def _agg_kernel_body(full, cpt, x_hbm, src_hbm, dst_hbm, zeros_hbm,
                     gidx_hbm, out_hbm, src_i, dst_i, *bufs):
    rows = bufs[:_NBUF]
    x_sp = bufs[_NBUF]
    acc = bufs[_NBUF + 1]
    sems = bufs[_NBUF + 2:]
    c = lax.axis_index("c")
    s = lax.axis_index("s")
    chunks_pg = cpt * _NS

    z0 = pl.multiple_of(s * _SLAB, 8)
    o0 = pl.multiple_of(c * _NROWS + s * _SLAB, 8)
    pltpu.sync_copy(x_hbm.at[pl.ds(o0, _SLAB)], x_sp.at[pl.ds(z0, _SLAB)])
    pltpu.sync_copy(zeros_hbm.at[pl.ds(z0, _SLAB)], acc.at[pl.ds(z0, _SLAB)])
    plsc.subcore_barrier()

    groups = cpt // _NBUF

    def body(g, carry):
        r0 = pl.multiple_of(c * chunks_pg + s * cpt + g * _NBUF, _NBUF)
        pltpu.sync_copy(src_hbm.at[pl.ds(r0, _NBUF)], src_i)
        pltpu.sync_copy(dst_hbm.at[pl.ds(r0, _NBUF)], dst_i)
        descs = [pltpu.async_copy(x_sp.at[src_i.at[b]], rows[b], sems[b])
                 for b in range(_NBUF)]
        for b in range(_NBUF):
            descs[b].wait()
            pltpu.sync_copy(rows[b], acc.at[dst_i.at[b]], add=True)
        return carry

    lax.fori_loop(0, groups, body, 0)
    plsc.subcore_barrier()

    if full:
        for j in range(5):
            cnt = _CH if j < 4 else _SLAB - 4 * _CH
            r0 = pl.multiple_of(s * _SLAB + j * _CH, 8)
            q0 = pl.multiple_of(c * _NROWS + s * _SLAB + j * _CH, 8)
            rb = rows[j % _NBUF]
            pltpu.sync_copy(acc.at[pl.ds(r0, cnt)], rb.at[pl.ds(0, cnt)])
            pltpu.sync_copy(rb.at[pl.ds(0, cnt)], out_hbm.at[pl.ds(q0, cnt)])
    else:
        g0 = pl.multiple_of(c * _G + s * _CH, 8)
        pltpu.sync_copy(gidx_hbm.at[pl.ds(g0, _CH)], src_i.at[0])
        pltpu.async_copy(acc.at[src_i.at[0]], rows[0], sems[0]).wait()
        pltpu.sync_copy(rows[0], out_hbm.at[pl.ds(g0, _CH)])


@functools.lru_cache(maxsize=None)
def _make_agg(cpt, full):
    mesh = plsc.VectorSubcoreMesh(core_axis_name="c", subcore_axis_name="s")
    out_rows = _NC * _NROWS if full else _NC * _G
    body = functools.partial(_agg_kernel_body, full, cpt)
    if full:
        def wrapped(x, src, dst, zeros, out, *scratch):
            body(x, src, dst, zeros, None, out, *scratch)
    else:
        wrapped = body
    return pl.kernel(
        wrapped,
        out_type=jax.ShapeDtypeStruct((out_rows, _DA), _DT),
        mesh=mesh,
        compiler_params=pltpu.CompilerParams(use_tc_tiling_on_sc=False),
        scratch_types=(
            [pltpu.VMEM((_NBUF, _CH), jnp.int32),
             pltpu.VMEM((_NBUF, _CH), jnp.int32)]
            + [pltpu.VMEM((_CH, _DA), _DT)] * _NBUF
            + [pltpu.VMEM_SHARED((_NROWS, _DA), _DT)] * 2
            + [pltpu.SemaphoreType.DMA] * _NBUF
        ),
    )


_H1_BLK = 1264


def _h1_body(a_ref, w_ref, o_ref):
    x = a_ref[...].astype(jnp.float32)
    deg = jnp.maximum(x[:, _D:_D + 1], 1.0)
    h = jnp.dot(x[:, :_D] / deg, w_ref[...], preferred_element_type=jnp.float32)
    h = jnp.maximum(h, 0.0)
    col = lax.broadcasted_iota(jnp.int32, (_H1_BLK, _DA - _D), 1)
    aug = jnp.where(col == 0, 1.0, 0.0)
    o_ref[...] = jnp.concatenate([h, aug], axis=1).astype(_DT)


def _h1_call(a, w):
    grid = (_NC * _NROWS) // _H1_BLK
    return pl.pallas_call(
        _h1_body,
        grid=(grid,),
        in_specs=[
            pl.BlockSpec((_H1_BLK, _DA), lambda i: (i, 0)),
            pl.BlockSpec((_D, _D), lambda i: (0, 0)),
        ],
        out_specs=pl.BlockSpec((_H1_BLK, _DA), lambda i: (i, 0)),
        out_shape=jax.ShapeDtypeStruct((_NC * _NROWS, _DA), _DT),
    )(a, w)


def _loss_body(p_ref, w_ref, o_ref):
    r = p_ref[...].astype(jnp.float32)
    deg = jnp.maximum(r[:, _D:_D + 1], 1.0)
    e = jnp.dot(r[:, :_D] / deg, w_ref[...], preferred_element_type=jnp.float32)
    pos1, neg1 = e[:_B], e[_B:_G]
    pos2, neg2 = e[_G:_G + _B], e[_G + _B:]
    pd = jnp.sum(jnp.abs(pos1 - pos2), axis=1, keepdims=True)
    na = jnp.sum(jnp.abs(pos1 - neg2), axis=1, keepdims=True)
    nb = jnp.sum(jnp.abs(neg1 - pos2), axis=1, keepdims=True)
    la = jnp.maximum(pd - na + 3.0, 0.0)
    lb = jnp.maximum(pd - nb + 3.0, 0.0)
    o_ref[0, 0] = (jnp.sum(la) + jnp.sum(lb)) / _B


def _loss_call(p, w):
    return pl.pallas_call(
        _loss_body,
        out_specs=pl.BlockSpec(memory_space=pltpu.SMEM),
        out_shape=jax.ShapeDtypeStruct((1, 1), jnp.float32),
    )(p, w)


def kernel(link, neg1, neg2, edge_index1, edge_index2, emb_table1, emb_table2,
           W1, W2):
    i32 = jnp.int32
    seed1 = link[:, 0].astype(i32)
    seed2 = link[:, 1].astype(i32)
    E = edge_index1.shape[1]
    cpt = -(-E // (_NS * _CH))
    cpt = -(-cpt // _NBUF) * _NBUF
    e_pad = _NS * _CH * cpt - E

    def prep_edges(ei):
        src = jnp.concatenate([ei[0].astype(i32), jnp.zeros((e_pad,), i32)])
        dst = jnp.concatenate([ei[1].astype(i32), jnp.full((e_pad,), _N, i32)])
        return src.reshape(-1, _CH), dst.reshape(-1, _CH)

    src1, dst1 = prep_edges(edge_index1)
    src2, dst2 = prep_edges(edge_index2)
    src = jnp.concatenate([src1, src2])
    dst = jnp.concatenate([dst1, dst2])
    zeros = jnp.zeros((_NROWS, _DA), _DT)

    def aug_table(t):
        return (jnp.zeros((_NROWS, _DA), jnp.float32)
                .at[:_N, :_D].set(t.astype(jnp.float32))
                .at[:_N, _D].set(1.0)).astype(_DT)

    x = jnp.concatenate([aug_table(emb_table1), aug_table(emb_table2)])
    gidx = jnp.concatenate([seed1, neg1.astype(i32),
                            seed2, neg2.astype(i32)])

    agg_full = _make_agg(cpt, True)
    agg_gather = _make_agg(cpt, False)

    a = agg_full(x, src, dst, zeros)
    h = _h1_call(a, W1)
    p = agg_gather(h, src, dst, zeros, gidx)
    return _loss_call(p, W2)[0, 0]
